# Initial kernel scaffold; baseline (speedup 1.0000x reference)
import jax
import jax.numpy as jnp
from jax.experimental import pallas as pl


def kernel(sequence, table):
    raise NotImplementedError("write your pallas kernel here")


# speedup vs baseline: 2.4344x; 2.4344x over previous
import functools

import numpy as np
import jax
import jax.numpy as jnp
from jax import lax
from jax.experimental import pallas as pl
from jax.experimental.pallas import tpu as pltpu
from jax.experimental.pallas import tpu_sc as plsc

B = 4096
L = 200
D = 64
B_TOTAL = B * L
NW = 32
PER_W = B_TOTAL // NW
SUB = 128
CHUNK = 512
NSUB = CHUNK // SUB
N_CHUNKS = PER_W // CHUNK
LANES = 16


def _positional_encoding():
    pos = np.arange(L, dtype=np.float32)[:, None]
    div = np.exp(np.arange(0, D, 2, dtype=np.float32) * (-np.log(10000.0) / D))
    pe = np.zeros((L, D), dtype=np.float32)
    pe[:, 0::2] = np.sin(pos * div)
    pe[:, 1::2] = np.cos(pos * div)
    return jnp.asarray(pe)


@functools.partial(
    pl.kernel,
    mesh=plsc.VectorSubcoreMesh(core_axis_name="c", subcore_axis_name="s"),
    compiler_params=pltpu.CompilerParams(use_tc_tiling_on_sc=False),
    out_type=jax.ShapeDtypeStruct((B_TOTAL, D), jnp.float32),
    scratch_types=[
        pltpu.VMEM((NSUB, SUB), jnp.int32),
        pltpu.VMEM((CHUNK, D), jnp.float32),
        pltpu.VMEM((L, D), jnp.float32),
        pltpu.SemaphoreType.DMA,
    ],
)
def _sc_embed(seq_hbm, pe_hbm, table_hbm, out_hbm, idx_v, rows_v, pe_v, sem):
    nc = lax.axis_size("c")
    wid = lax.axis_index("s") * nc + lax.axis_index("c")
    pltpu.sync_copy(pe_hbm, pe_v)

    def chunk_body(c, pos0):
        base = wid * PER_W + c * CHUNK
        idx_row = wid * (PER_W // SUB) + c * NSUB
        pltpu.sync_copy(seq_hbm.at[pl.ds(idx_row, NSUB)], idx_v)
        cps = [
            pltpu.async_copy(
                table_hbm.at[idx_v.at[j]], rows_v.at[pl.ds(j * SUB, SUB)], sem
            )
            for j in range(NSUB)
        ]
        for cp in cps:
            cp.wait()

        def row_body(r, pos):
            for dd in range(D // LANES):
                sl = pl.ds(dd * LANES, LANES)
                rows_v[r, sl] += pe_v[pos, sl]
            nxt = pos + 1
            return lax.select(nxt == L, 0, nxt)

        pos_end = lax.fori_loop(0, CHUNK, row_body, pos0)
        pltpu.sync_copy(rows_v, out_hbm.at[pl.ds(base, CHUNK)])
        return pos_end

    lax.fori_loop(0, N_CHUNKS, chunk_body, jnp.int32(0))


def kernel(sequence, table):
    assert sequence.shape == (B, L), sequence.shape
    assert table.shape[1] == D, table.shape
    seq2d = sequence.reshape(B_TOTAL // SUB, SUB).astype(jnp.int32)
    pe = _positional_encoding()
    out = _sc_embed(seq2d, pe, table)
    return out.reshape(B, L, D)


# speedup vs baseline: 2.7416x; 1.1262x over previous
import functools

import numpy as np
import jax
import jax.numpy as jnp
from jax import lax
from jax.experimental import pallas as pl
from jax.experimental.pallas import tpu as pltpu
from jax.experimental.pallas import tpu_sc as plsc

B = 4096
L = 200
D = 64
B_TOTAL = B * L
NW = 32
PER_W = B_TOTAL // NW
SUB = 128
CHUNK = 512
NSUB = CHUNK // SUB
N_CHUNKS = PER_W // CHUNK
PAIRS = N_CHUNKS // 2
IDX_ROWS = PER_W // SUB
LANES = 16


def _positional_encoding():
    pos = np.arange(L, dtype=np.float32)[:, None]
    div = np.exp(np.arange(0, D, 2, dtype=np.float32) * (-np.log(10000.0) / D))
    pe = np.zeros((L, D), dtype=np.float32)
    pe[:, 0::2] = np.sin(pos * div)
    pe[:, 1::2] = np.cos(pos * div)
    return jnp.asarray(pe)


@functools.partial(
    pl.kernel,
    mesh=plsc.VectorSubcoreMesh(core_axis_name="c", subcore_axis_name="s"),
    compiler_params=pltpu.CompilerParams(use_tc_tiling_on_sc=False),
    out_type=jax.ShapeDtypeStruct((B_TOTAL, D), jnp.float32),
    scratch_types=[
        pltpu.VMEM((IDX_ROWS + NSUB, SUB), jnp.int32),
        pltpu.VMEM((CHUNK, D), jnp.float32),
        pltpu.VMEM((CHUNK, D), jnp.float32),
        pltpu.VMEM((L, D), jnp.float32),
        pltpu.SemaphoreType.DMA,
        pltpu.SemaphoreType.DMA,
    ],
)
def _sc_embed(seq_hbm, pe_hbm, table_hbm, out_hbm,
              idx_v, rows0, rows1, pe_v, sg0, sg1):
    nc = lax.axis_size("c")
    wid = lax.axis_index("s") * nc + lax.axis_index("c")
    pltpu.sync_copy(pe_hbm, pe_v)
    pltpu.sync_copy(seq_hbm.at[pl.ds(wid * IDX_ROWS, IDX_ROWS + NSUB)], idx_v)

    def issue_gather(c, rows_ref, sem):
        for j in range(NSUB):
            pltpu.async_copy(
                table_hbm.at[idx_v.at[c * NSUB + j]],
                rows_ref.at[pl.ds(j * SUB, SUB)],
                sem,
            )

    def wait_gather(rows_ref, sem):
        pltpu.make_async_copy(out_hbm.at[pl.ds(0, CHUNK)], rows_ref, sem).wait()

    def compute(rows_ref, pos0):
        def row_body(r, pos):
            for dd in range(D // LANES):
                sl = pl.ds(dd * LANES, LANES)
                rows_ref[r, sl] += pe_v[pos, sl]
            nxt = pos + 1
            return lax.select(nxt == L, 0, nxt)

        return lax.fori_loop(0, CHUNK, row_body, pos0, unroll=8)

    def step(c, buf, nbuf, sem, nsem, pos):
        issue_gather(c + 1, nbuf, nsem)
        wait_gather(buf, sem)
        pos = compute(buf, pos)
        pltpu.sync_copy(buf, out_hbm.at[pl.ds(wid * PER_W + c * CHUNK, CHUNK)])
        return pos

    issue_gather(0, rows0, sg0)

    def pair_body(g, pos):
        pos = step(2 * g, rows0, rows1, sg0, sg1, pos)
        pos = step(2 * g + 1, rows1, rows0, sg1, sg0, pos)
        return pos

    lax.fori_loop(0, PAIRS, pair_body, jnp.int32(0))
    wait_gather(rows0, sg0)


def kernel(sequence, table):
    assert sequence.shape == (B, L), sequence.shape
    assert table.shape[1] == D, table.shape
    seq2d = sequence.reshape(B_TOTAL // SUB, SUB).astype(jnp.int32)
    seq2d = jnp.concatenate(
        [seq2d, jnp.zeros((NSUB, SUB), jnp.int32)], axis=0
    )
    pe = _positional_encoding()
    out = _sc_embed(seq2d, pe, table)
    return out.reshape(B, L, D)


# speedup vs baseline: 3.3814x; 1.2334x over previous
import functools

import numpy as np
import jax
import jax.numpy as jnp
from jax import lax
from jax.experimental import pallas as pl
from jax.experimental.pallas import tpu as pltpu
from jax.experimental.pallas import tpu_sc as plsc

B = 4096
L = 200
D = 64
NW = 32
SEQ_PER_W = B // NW
SEQ_PER_CHUNK = 2
CHUNK = SEQ_PER_CHUNK * L
N_CHUNKS = SEQ_PER_W // SEQ_PER_CHUNK
PAIRS = N_CHUNKS // 2
SPLITS = ((0, 104), (104, 96))
LANES = 16


def _positional_encoding():
    pos = np.arange(L, dtype=np.float32)[:, None]
    div = np.exp(np.arange(0, D, 2, dtype=np.float32) * (-np.log(10000.0) / D))
    pe = np.zeros((L, D), dtype=np.float32)
    pe[:, 0::2] = np.sin(pos * div)
    pe[:, 1::2] = np.cos(pos * div)
    return jnp.asarray(pe)


@functools.partial(
    pl.kernel,
    mesh=plsc.VectorSubcoreMesh(core_axis_name="c", subcore_axis_name="s"),
    compiler_params=pltpu.CompilerParams(use_tc_tiling_on_sc=False),
    out_type=jax.ShapeDtypeStruct((B, L, D), jnp.float32),
    scratch_types=[
        pltpu.VMEM((SEQ_PER_W, L), jnp.int32),
        pltpu.VMEM((SEQ_PER_CHUNK, L, D), jnp.float32),
        pltpu.VMEM((SEQ_PER_CHUNK, L, D), jnp.float32),
        pltpu.VMEM((L, D), jnp.float32),
        pltpu.SemaphoreType.DMA,
        pltpu.SemaphoreType.DMA,
    ],
)
def _sc_embed(seq_hbm, pe_hbm, table_hbm, out_hbm,
              idx_v, rows0, rows1, pe_v, sg0, sg1):
    nc = lax.axis_size("c")
    wid = lax.axis_index("s") * nc + lax.axis_index("c")
    seq0 = wid * SEQ_PER_W
    pltpu.sync_copy(pe_hbm, pe_v)
    pltpu.sync_copy(seq_hbm.at[pl.ds(seq0, SEQ_PER_W)], idx_v)

    def issue_gather(c, rows_ref, sem):
        for s in range(SEQ_PER_CHUNK):
            for off, n in SPLITS:
                pltpu.async_copy(
                    table_hbm.at[idx_v.at[c * SEQ_PER_CHUNK + s, pl.ds(off, n)]],
                    rows_ref.at[s, pl.ds(off, n)],
                    sem,
                )

    def wait_gather(rows_ref, sem):
        pltpu.make_async_copy(
            out_hbm.at[pl.ds(0, SEQ_PER_CHUNK)], rows_ref, sem
        ).wait()

    def compute(rows_ref):
        def row_body(r, carry):
            for dd in range(D // LANES):
                sl = pl.ds(dd * LANES, LANES)
                pe_vec = pe_v[r, sl]
                for s in range(SEQ_PER_CHUNK):
                    rows_ref[s, r, sl] += pe_vec
            return carry

        lax.fori_loop(0, L, row_body, 0, unroll=8)

    def step(c, buf, nbuf, sem, nsem):
        issue_gather(lax.min(c + 1, N_CHUNKS - 1), nbuf, nsem)
        wait_gather(buf, sem)
        compute(buf)
        pltpu.sync_copy(
            buf, out_hbm.at[pl.ds(seq0 + c * SEQ_PER_CHUNK, SEQ_PER_CHUNK)]
        )

    issue_gather(0, rows0, sg0)

    def pair_body(g, carry):
        step(2 * g, rows0, rows1, sg0, sg1)
        step(2 * g + 1, rows1, rows0, sg1, sg0)
        return carry

    lax.fori_loop(0, PAIRS, pair_body, 0)
    wait_gather(rows0, sg0)


def kernel(sequence, table):
    assert sequence.shape == (B, L), sequence.shape
    assert table.shape[1] == D, table.shape
    pe = _positional_encoding()
    return _sc_embed(sequence.astype(jnp.int32), pe, table)
